# baseline (speedup 1.0000x reference)
import jax
import jax.numpy as jnp
from jax.experimental import pallas as pl
from jax.experimental.pallas import tpu as pltpu

_EPS = 1e-06
_S = 0.025
_ALPHA = 0.98
_DELTA = 2.0
_R = 0.5

_CHUNK = 128


def _pcen_body(x_ref, o_ref, l_ref, d_ref, carry_ref, m_ref):
    t = pl.program_id(0)
    c = _CHUNK

    @pl.when(t == 0)
    def _init():
        decay = 1.0 - _S
        j = jax.lax.broadcasted_iota(jnp.int32, (c, c), 0).astype(jnp.float32)
        k = jax.lax.broadcasted_iota(jnp.int32, (c, c), 1).astype(jnp.float32)
        lmat = jnp.where(
            j >= k,
            _S * jnp.exp2((j - k) * jnp.log2(decay)),
            0.0,
        )
        l_ref[...] = lmat.astype(jnp.bfloat16)
        jj = jax.lax.broadcasted_iota(jnp.int32, (c, 1), 0).astype(jnp.float32)
        d_ref[...] = jnp.exp2((jj + 1.0) * jnp.log2(decay))
        carry_ref[...] = x_ref[:, 0, :]

    lmat = l_ref[...]
    dvec = d_ref[...]

    def body(i, _):
        m = jax.lax.dot(
            lmat, x_ref[i].astype(jnp.bfloat16),
            preferred_element_type=jnp.float32,
        ) + dvec * carry_ref[pl.ds(i, 1), :]
        carry_ref[pl.ds(i, 1), :] = m[c - 1:c, :]
        m_ref[i] = m
        return ()

    jax.lax.fori_loop(0, x_ref.shape[0], body, (), unroll=8)

    x = x_ref[...]
    m = m_ref[...]
    o_ref[...] = jnp.sqrt(
        x * jnp.exp(-_ALPHA * jnp.log(m + _EPS)) + _DELTA
    ) - _DELTA**_R


def _pcen_pallas(x):
    b, t, f = x.shape
    c = _CHUNK
    return pl.pallas_call(
        _pcen_body,
        grid=(t // c,),
        in_specs=[pl.BlockSpec((b, c, f), lambda ti: (0, ti, 0))],
        out_specs=pl.BlockSpec((b, c, f), lambda ti: (0, ti, 0)),
        out_shape=jax.ShapeDtypeStruct((b, t, f), jnp.float32),
        scratch_shapes=[
            pltpu.VMEM((c, c), jnp.bfloat16),
            pltpu.VMEM((c, 1), jnp.float32),
            pltpu.VMEM((b, f), jnp.float32),
            pltpu.VMEM((b, c, f), jnp.float32),
        ],
        compiler_params=pltpu.CompilerParams(
            dimension_semantics=("arbitrary",),
        ),
    )(x)


def kernel(x):
    return _pcen_pallas(x)


# speedup vs baseline: 1.0233x; 1.0233x over previous
import jax
import jax.numpy as jnp
from jax.experimental import pallas as pl
from jax.experimental.pallas import tpu as pltpu

_EPS = 1e-06
_S = 0.025
_ALPHA = 0.98
_DELTA = 2.0
_R = 0.5

_CHUNK = 256


def _pcen_body(x_ref, o_ref, l_ref, d_ref, carry_ref, m_ref):
    t = pl.program_id(0)
    c = _CHUNK

    @pl.when(t == 0)
    def _init():
        decay = 1.0 - _S
        j = jax.lax.broadcasted_iota(jnp.int32, (c, c), 0).astype(jnp.float32)
        k = jax.lax.broadcasted_iota(jnp.int32, (c, c), 1).astype(jnp.float32)
        lmat = jnp.where(
            j >= k,
            _S * jnp.exp2((j - k) * jnp.log2(decay)),
            0.0,
        )
        l_ref[...] = lmat.astype(jnp.bfloat16)
        jj = jax.lax.broadcasted_iota(jnp.int32, (c, 1), 0).astype(jnp.float32)
        d_ref[...] = jnp.exp2((jj + 1.0) * jnp.log2(decay))
        carry_ref[...] = x_ref[:, 0, :]

    lmat = l_ref[...]
    dvec = d_ref[...]

    b = x_ref.shape[0]
    g = 8
    for base in range(0, b, g):
        for i in range(base, base + g):
            m = jax.lax.dot(
                lmat, x_ref[i].astype(jnp.bfloat16),
                preferred_element_type=jnp.float32,
            ) + dvec * carry_ref[i:i + 1, :]
            carry_ref[i:i + 1, :] = m[c - 1:c, :]
            m_ref[i] = m
        xg = x_ref[base:base + g]
        mg = m_ref[base:base + g]
        o_ref[base:base + g] = jnp.sqrt(
            xg * jnp.exp(-_ALPHA * jnp.log(mg + _EPS)) + _DELTA
        ) - _DELTA**_R


def _pcen_pallas(x):
    b, t, f = x.shape
    c = _CHUNK
    return pl.pallas_call(
        _pcen_body,
        grid=(t // c,),
        in_specs=[pl.BlockSpec((b, c, f), lambda ti: (0, ti, 0))],
        out_specs=pl.BlockSpec((b, c, f), lambda ti: (0, ti, 0)),
        out_shape=jax.ShapeDtypeStruct((b, t, f), jnp.float32),
        scratch_shapes=[
            pltpu.VMEM((c, c), jnp.bfloat16),
            pltpu.VMEM((c, 1), jnp.float32),
            pltpu.VMEM((b, f), jnp.float32),
            pltpu.VMEM((b, c, f), jnp.float32),
        ],
        compiler_params=pltpu.CompilerParams(
            dimension_semantics=("arbitrary",),
        ),
    )(x)


def kernel(x):
    return _pcen_pallas(x)


# speedup vs baseline: 1.0250x; 1.0017x over previous
import jax
import jax.numpy as jnp
from jax.experimental import pallas as pl
from jax.experimental.pallas import tpu as pltpu

_EPS = 1e-06
_S = 0.025
_ALPHA = 0.98
_DELTA = 2.0
_R = 0.5

_CHUNK = 256


def _out_copy(obuf_ref, o_hbm_ref, sem, slot, t, c):
    return pltpu.make_async_copy(
        obuf_ref.at[slot],
        o_hbm_ref.at[:, pl.ds(t * c, c), :],
        sem.at[slot],
    )


def _pcen_body(x_ref, o_ref, l_ref, d_ref, carry_ref, m_ref, obuf_ref, sem):
    t = pl.program_id(0)
    nsteps = pl.num_programs(0)
    c = _CHUNK

    @pl.when(t == 0)
    def _init():
        decay = 1.0 - _S
        j = jax.lax.broadcasted_iota(jnp.int32, (c, c), 0).astype(jnp.float32)
        k = jax.lax.broadcasted_iota(jnp.int32, (c, c), 1).astype(jnp.float32)
        lmat = jnp.where(
            j >= k,
            _S * jnp.exp2((j - k) * jnp.log2(decay)),
            0.0,
        )
        l_ref[...] = lmat.astype(jnp.bfloat16)
        jj = jax.lax.broadcasted_iota(jnp.int32, (c, 1), 0).astype(jnp.float32)
        d_ref[...] = jnp.exp2((jj + 1.0) * jnp.log2(decay))
        carry_ref[...] = x_ref[:, 0, :]

    slot = jax.lax.rem(t, 2)

    @pl.when(t >= 2)
    def _reclaim():
        _out_copy(obuf_ref, o_ref, sem, slot, t - 2, c).wait()

    lmat = l_ref[...]
    dvec = d_ref[...]

    b = x_ref.shape[0]
    g = 8
    for base in range(0, b, g):
        for i in range(base, base + g):
            m = jax.lax.dot(
                lmat, x_ref[i].astype(jnp.bfloat16),
                preferred_element_type=jnp.float32,
            ) + dvec * carry_ref[i:i + 1, :]
            carry_ref[i:i + 1, :] = m[c - 1:c, :]
            m_ref[i] = m
        xg = x_ref[base:base + g]
        mg = m_ref[base:base + g]
        obuf_ref[slot, base:base + g] = jnp.sqrt(
            xg * jnp.exp(-_ALPHA * jnp.log(mg + _EPS)) + _DELTA
        ) - _DELTA**_R

    _out_copy(obuf_ref, o_ref, sem, slot, t, c).start()

    @pl.when(t == nsteps - 1)
    def _drain():
        _out_copy(obuf_ref, o_ref, sem, 1 - slot, t - 1, c).wait()
        _out_copy(obuf_ref, o_ref, sem, slot, t, c).wait()


def _pcen_pallas(x):
    b, t, f = x.shape
    c = _CHUNK
    return pl.pallas_call(
        _pcen_body,
        grid=(t // c,),
        in_specs=[pl.BlockSpec((b, c, f), lambda ti: (0, ti, 0))],
        out_specs=pl.BlockSpec(memory_space=pltpu.MemorySpace.HBM),
        out_shape=jax.ShapeDtypeStruct((b, t, f), jnp.float32),
        scratch_shapes=[
            pltpu.VMEM((c, c), jnp.bfloat16),
            pltpu.VMEM((c, 1), jnp.float32),
            pltpu.VMEM((b, f), jnp.float32),
            pltpu.VMEM((b, c, f), jnp.float32),
            pltpu.VMEM((2, b, c, f), jnp.float32),
            pltpu.SemaphoreType.DMA((2,)),
        ],
        compiler_params=pltpu.CompilerParams(
            dimension_semantics=("arbitrary",),
        ),
    )(x)


def kernel(x):
    return _pcen_pallas(x)


# speedup vs baseline: 1.0685x; 1.0425x over previous
import jax
import jax.numpy as jnp
from jax.experimental import pallas as pl
from jax.experimental.pallas import tpu as pltpu

_EPS = 1e-06
_S = 0.025
_ALPHA = 0.98
_DELTA = 2.0
_R = 0.5

_CHUNK = 256


def _out_copy(obuf_ref, o_hbm_ref, sem, slot, t, c):
    return pltpu.make_async_copy(
        obuf_ref.at[slot],
        o_hbm_ref.at[:, pl.ds(t * c, c), :],
        sem.at[slot],
    )


def _pcen_body(x_ref, o_ref, l_ref, d_ref, carry_ref, m_ref, obuf_ref, sem):
    t = pl.program_id(0)
    nsteps = pl.num_programs(0)
    c = _CHUNK

    @pl.when(t == 0)
    def _init():
        decay = 1.0 - _S
        j = jax.lax.broadcasted_iota(jnp.int32, (c, c), 0).astype(jnp.float32)
        k = jax.lax.broadcasted_iota(jnp.int32, (c, c), 1).astype(jnp.float32)
        lmat = jnp.where(
            j >= k,
            _S * jnp.exp2((j - k) * jnp.log2(decay)),
            0.0,
        )
        l_ref[...] = lmat.astype(jnp.bfloat16)
        jj = jax.lax.broadcasted_iota(jnp.int32, (c, 1), 0).astype(jnp.float32)
        d_ref[...] = jnp.exp2((jj + 1.0) * jnp.log2(decay))
        carry_ref[...] = x_ref[:, 0, :]

    slot = jax.lax.rem(t, 2)

    @pl.when(t >= 2)
    def _reclaim():
        _out_copy(obuf_ref, o_ref, sem, slot, t - 2, c).wait()

    lmat = l_ref[...]
    dvec = d_ref[...]

    b = x_ref.shape[0]
    g = 8
    for base in range(0, b, g):
        xg = x_ref[base:base + g]
        ms = []
        for i in range(g):
            m = jax.lax.dot(
                lmat, xg[i].astype(jnp.bfloat16),
                preferred_element_type=jnp.float32,
            ) + dvec * carry_ref[base + i:base + i + 1, :]
            carry_ref[base + i:base + i + 1, :] = m[c - 1:c, :]
            ms.append(m)
        mg = jnp.stack(ms, axis=0)
        obuf_ref[slot, base:base + g] = jnp.sqrt(
            xg * jnp.exp2(jnp.log2(mg + _EPS) * -_ALPHA) + _DELTA
        ) - _DELTA**_R

    _out_copy(obuf_ref, o_ref, sem, slot, t, c).start()

    @pl.when(t == nsteps - 1)
    def _drain():
        _out_copy(obuf_ref, o_ref, sem, 1 - slot, t - 1, c).wait()
        _out_copy(obuf_ref, o_ref, sem, slot, t, c).wait()


def _pcen_pallas(x):
    b, t, f = x.shape
    c = _CHUNK
    return pl.pallas_call(
        _pcen_body,
        grid=(t // c,),
        in_specs=[pl.BlockSpec((b, c, f), lambda ti: (0, ti, 0))],
        out_specs=pl.BlockSpec(memory_space=pltpu.MemorySpace.HBM),
        out_shape=jax.ShapeDtypeStruct((b, t, f), jnp.float32),
        scratch_shapes=[
            pltpu.VMEM((c, c), jnp.bfloat16),
            pltpu.VMEM((c, 1), jnp.float32),
            pltpu.VMEM((b, f), jnp.float32),
            pltpu.VMEM((b, c, f), jnp.float32),
            pltpu.VMEM((2, b, c, f), jnp.float32),
            pltpu.SemaphoreType.DMA((2,)),
        ],
        compiler_params=pltpu.CompilerParams(
            dimension_semantics=("arbitrary",),
        ),
    )(x)


def kernel(x):
    return _pcen_pallas(x)


# speedup vs baseline: 1.0698x; 1.0012x over previous
import jax
import jax.numpy as jnp
from jax.experimental import pallas as pl
from jax.experimental.pallas import tpu as pltpu

_EPS = 1e-06
_S = 0.025
_ALPHA = 0.98
_DELTA = 2.0
_R = 0.5

_CHUNK = 256


def _out_copy(obuf_ref, o_hbm_ref, sem, slot, t, c):
    return pltpu.make_async_copy(
        obuf_ref.at[slot],
        o_hbm_ref.at[:, pl.ds(t * c, c), :],
        sem.at[slot],
    )


def _pcen_body(x_ref, o_ref, l_ref, d_ref, carry_ref, obuf_ref, sem):
    t = pl.program_id(0)
    nsteps = pl.num_programs(0)
    c = _CHUNK

    @pl.when(t == 0)
    def _init():
        decay = 1.0 - _S
        j = jax.lax.broadcasted_iota(jnp.int32, (c, c), 0).astype(jnp.float32)
        k = jax.lax.broadcasted_iota(jnp.int32, (c, c), 1).astype(jnp.float32)
        lmat = jnp.where(
            j >= k,
            _S * jnp.exp2((j - k) * jnp.log2(decay)),
            0.0,
        )
        l_ref[...] = lmat.astype(jnp.bfloat16)
        jj = jax.lax.broadcasted_iota(jnp.int32, (c, 1), 0).astype(jnp.float32)
        d_ref[...] = jnp.exp2((jj + 1.0) * jnp.log2(decay))
        carry_ref[...] = x_ref[:, 0, :]

    slot = jax.lax.rem(t, 2)

    @pl.when(t >= 2)
    def _reclaim():
        _out_copy(obuf_ref, o_ref, sem, slot, t - 2, c).wait()

    lmat = l_ref[...]
    dvec = d_ref[...]

    b = x_ref.shape[0]
    g = 16
    for base in range(0, b, g):
        xg = x_ref[base:base + g]
        ms = []
        for i in range(g):
            m = jax.lax.dot(
                lmat, xg[i].astype(jnp.bfloat16),
                preferred_element_type=jnp.float32,
            ) + dvec * carry_ref[base + i:base + i + 1, :]
            carry_ref[base + i:base + i + 1, :] = m[c - 1:c, :]
            ms.append(m)
        mg = jnp.stack(ms, axis=0)
        obuf_ref[slot, base:base + g] = jnp.sqrt(
            xg * jnp.exp2(jnp.log2(mg + _EPS) * -_ALPHA) + _DELTA
        ) - _DELTA**_R

    _out_copy(obuf_ref, o_ref, sem, slot, t, c).start()

    @pl.when(t == nsteps - 1)
    def _drain():
        _out_copy(obuf_ref, o_ref, sem, 1 - slot, t - 1, c).wait()
        _out_copy(obuf_ref, o_ref, sem, slot, t, c).wait()


def _pcen_pallas(x):
    b, t, f = x.shape
    c = _CHUNK
    return pl.pallas_call(
        _pcen_body,
        grid=(t // c,),
        in_specs=[pl.BlockSpec((b, c, f), lambda ti: (0, ti, 0))],
        out_specs=pl.BlockSpec(memory_space=pltpu.MemorySpace.HBM),
        out_shape=jax.ShapeDtypeStruct((b, t, f), jnp.float32),
        scratch_shapes=[
            pltpu.VMEM((c, c), jnp.bfloat16),
            pltpu.VMEM((c, 1), jnp.float32),
            pltpu.VMEM((b, f), jnp.float32),
            pltpu.VMEM((2, b, c, f), jnp.float32),
            pltpu.SemaphoreType.DMA((2,)),
        ],
        compiler_params=pltpu.CompilerParams(
            dimension_semantics=("arbitrary",),
        ),
    )(x)


def kernel(x):
    return _pcen_pallas(x)


# speedup vs baseline: 1.0777x; 1.0074x over previous
import jax
import jax.numpy as jnp
from jax.experimental import pallas as pl
from jax.experimental.pallas import tpu as pltpu

_EPS = 1e-06
_S = 0.025
_ALPHA = 0.98
_DELTA = 2.0
_R = 0.5

_CHUNK = 256


def _out_copy(obuf_ref, o_hbm_ref, sem, slot, t, c, base, g):
    return pltpu.make_async_copy(
        obuf_ref.at[slot, pl.ds(base, g)],
        o_hbm_ref.at[pl.ds(base, g), pl.ds(t * c, c), :],
        sem.at[slot],
    )


def _pcen_body(x_ref, o_ref, l_ref, d_ref, carry_ref, obuf_ref, sem):
    t = pl.program_id(0)
    nsteps = pl.num_programs(0)
    c = _CHUNK

    @pl.when(t == 0)
    def _init():
        decay = 1.0 - _S
        j = jax.lax.broadcasted_iota(jnp.int32, (c, c), 0).astype(jnp.float32)
        k = jax.lax.broadcasted_iota(jnp.int32, (c, c), 1).astype(jnp.float32)
        lmat = jnp.where(
            j >= k,
            _S * jnp.exp2((j - k) * jnp.log2(decay)),
            0.0,
        )
        l_ref[...] = lmat.astype(jnp.bfloat16)
        jj = jax.lax.broadcasted_iota(jnp.int32, (c, 1), 0).astype(jnp.float32)
        d_ref[...] = jnp.exp2((jj + 1.0) * jnp.log2(decay))
        carry_ref[...] = x_ref[:, 0, :]

    b = x_ref.shape[0]
    g = 16

    slot = jax.lax.rem(t, 2)

    @pl.when(t >= 2)
    def _reclaim():
        for base in range(0, b, g):
            _out_copy(obuf_ref, o_ref, sem, slot, t - 2, c, base, g).wait()

    lmat = l_ref[...]
    dvec = d_ref[...]

    for base in range(0, b, g):
        xg = x_ref[base:base + g]
        ms = []
        for i in range(g):
            m = jax.lax.dot(
                lmat, xg[i].astype(jnp.bfloat16),
                preferred_element_type=jnp.float32,
            ) + dvec * carry_ref[base + i:base + i + 1, :]
            carry_ref[base + i:base + i + 1, :] = m[c - 1:c, :]
            ms.append(m)
        mg = jnp.stack(ms, axis=0)
        obuf_ref[slot, base:base + g] = jnp.sqrt(
            xg * jnp.exp2(jnp.log2(mg + _EPS) * -_ALPHA) + _DELTA
        ) - _DELTA**_R
        _out_copy(obuf_ref, o_ref, sem, slot, t, c, base, g).start()

    @pl.when(t == nsteps - 1)
    def _drain():
        for base in range(0, b, g):
            _out_copy(obuf_ref, o_ref, sem, 1 - slot, t - 1, c, base, g).wait()
        for base in range(0, b, g):
            _out_copy(obuf_ref, o_ref, sem, slot, t, c, base, g).wait()


def _pcen_pallas(x):
    b, t, f = x.shape
    c = _CHUNK
    return pl.pallas_call(
        _pcen_body,
        grid=(t // c,),
        in_specs=[pl.BlockSpec((b, c, f), lambda ti: (0, ti, 0))],
        out_specs=pl.BlockSpec(memory_space=pltpu.MemorySpace.HBM),
        out_shape=jax.ShapeDtypeStruct((b, t, f), jnp.float32),
        scratch_shapes=[
            pltpu.VMEM((c, c), jnp.bfloat16),
            pltpu.VMEM((c, 1), jnp.float32),
            pltpu.VMEM((b, f), jnp.float32),
            pltpu.VMEM((2, b, c, f), jnp.float32),
            pltpu.SemaphoreType.DMA((2,)),
        ],
        compiler_params=pltpu.CompilerParams(
            dimension_semantics=("arbitrary",),
        ),
    )(x)


def kernel(x):
    return _pcen_pallas(x)
